# Initial kernel scaffold; baseline (speedup 1.0000x reference)
import jax
import jax.numpy as jnp
from jax.experimental import pallas as pl


def kernel(xyz1, xyz2, points1, points2, percentage, direction, features):
    raise NotImplementedError("write your pallas kernel here")


# speedup vs baseline: 13.7977x; 13.7977x over previous
import functools

import jax
import jax.numpy as jnp
from jax import lax
from jax.experimental import pallas as pl
from jax.experimental.pallas import tpu as pltpu
from jax.experimental.pallas import tpu_sc as plsc

N = 16384
S = 4096
D = 64
K = 20
NB = 8
GAMMA = 0.5
ROW = 256

T1 = 256
T3 = 256
T4 = 2048

NUM_SC = 2
NUM_SUBCORES = 16
NW = NUM_SC * NUM_SUBCORES
B_IDX = NB * N
B_PER_W = B_IDX // NW
CHUNK = 256


def _topk_body(d_ref, idx_ref, s_ref):
    s_ref[...] = d_ref[...]
    col = lax.broadcasted_iota(jnp.int32, (T1, S), 1)
    for j in range(NB):
        s = s_ref[...]
        mn = jnp.min(s, axis=1, keepdims=True)
        am = jnp.min(jnp.where(s <= mn, col, S), axis=1, keepdims=True)
        idx_ref[:, j:j + 1] = am
        s_ref[...] = jnp.where(col == am, jnp.inf, s)


def _topk_idx(dists):
    return pl.pallas_call(
        _topk_body,
        grid=(N // T1,),
        in_specs=[pl.BlockSpec((T1, S), lambda i: (i, 0))],
        out_specs=pl.BlockSpec((T1, NB), lambda i: (i, 0)),
        out_shape=jax.ShapeDtypeStruct((N, NB), jnp.int32),
        scratch_shapes=[pltpu.VMEM((T1, S), jnp.float32)],
    )(dists)


def _sc_gather(table, idx_flat):
    mesh = plsc.VectorSubcoreMesh(core_axis_name="c", subcore_axis_name="s")

    @functools.partial(
        pl.kernel,
        mesh=mesh,
        out_type=jax.ShapeDtypeStruct((B_IDX, ROW), jnp.float32),
        scratch_types=[
            pltpu.VMEM((CHUNK,), jnp.int32),
            pltpu.VMEM((CHUNK, ROW), jnp.float32),
            pltpu.SemaphoreType.DMA,
        ],
    )
    def gather_kernel(table_hbm, idx_hbm, out_hbm, idx_v, rows_v, sem):
        wid = lax.axis_index("s") * NUM_SC + lax.axis_index("c")
        base = wid * B_PER_W

        @pl.loop(0, B_PER_W, step=CHUNK)
        def _(c):
            off = base + c
            pltpu.sync_copy(idx_hbm.at[pl.ds(off, CHUNK)], idx_v)
            pltpu.async_copy(table_hbm.at[idx_v], rows_v, sem).wait()
            pltpu.sync_copy(rows_v, out_hbm.at[pl.ds(off, CHUNK)])

    return gather_kernel(table, idx_flat)


def _halve_sum(parts):
    parts = list(parts)
    while len(parts) > 1:
        h = len(parts) // 2
        parts = [parts[i] + parts[h + i] for i in range(h)]
    return parts[0]


def _interp_body(g_ref, x1_ref, interp_ref, wsum_ref):
    x1 = x1_ref[...]
    wk_cols = []
    for n in range(NB):
        g = g_ref[n]
        ux = g[:, 0:1] - x1[:, 0:1]
        uy = g[:, 1:2] - x1[:, 1:2]
        uz = g[:, 2:3] - x1[:, 2:3]
        dist = jnp.sqrt(ux * ux + uy * uy + uz * uz)
        den = dist + 1e-08
        ux, uy, uz = ux / den, uy / den, uz / den
        d0 = g[:, 23:43]
        d1 = g[:, 43:63]
        d2 = g[:, 63:83]
        dn = jnp.sqrt(d0 * d0 + d1 * d1 + d2 * d2) + 1e-08
        simm = (d0 / dn) * ux + (d1 / dn) * uy + (d2 / dn) * uz
        mask = jnp.abs(simm) > GAMMA
        t = jnp.where(mask, g[:, 3:23], 0.0)
        wk_n = t[:, 0:1]
        for k in range(1, K):
            wk_n = wk_n + t[:, k:k + 1]
        wk_cols.append(wk_n)
    wsum = _halve_sum(wk_cols) + 1e-08
    dr_cols = [wk_n / wsum + 1e-06 + 1e-10 for wk_n in wk_cols]
    norm = _halve_sum(dr_cols) + 1e-08
    acc = jnp.zeros((T3, D), jnp.float32)
    for n in range(NB):
        acc = acc + g_ref[n][:, 83:83 + D] * (dr_cols[n] / norm)
    interp_ref[...] = acc
    wsum_ref[...] = wsum


def _interpolate(g3, x1_2d):
    return pl.pallas_call(
        _interp_body,
        grid=(N // T3,),
        in_specs=[
            pl.BlockSpec((NB, T3, ROW), lambda i: (0, i, 0)),
            pl.BlockSpec((T3, 3), lambda i: (i, 0)),
        ],
        out_specs=[
            pl.BlockSpec((T3, D), lambda i: (i, 0)),
            pl.BlockSpec((T3, 1), lambda i: (i, 0)),
        ],
        out_shape=[
            jax.ShapeDtypeStruct((N, D), jnp.float32),
            jax.ShapeDtypeStruct((N, 1), jnp.float32),
        ],
    )(g3, x1_2d)


def _combine_body(interp_ref, wsum_ref, p1_ref, cs_ref, out_ref):
    out_ref[...] = (interp_ref[...] * wsum_ref[...]
                    + cs_ref[0, 0] * p1_ref[...])


def _combine(interp, wsum, p1_2d, cs):
    return pl.pallas_call(
        _combine_body,
        grid=(N // T4,),
        in_specs=[
            pl.BlockSpec((T4, D), lambda i: (i, 0)),
            pl.BlockSpec((T4, 1), lambda i: (i, 0)),
            pl.BlockSpec((T4, D), lambda i: (i, 0)),
            pl.BlockSpec(memory_space=pltpu.SMEM),
        ],
        out_specs=pl.BlockSpec((T4, D), lambda i: (i, 0)),
        out_shape=jax.ShapeDtypeStruct((N, D), jnp.float32),
    )(interp, wsum, p1_2d, cs)


def kernel(xyz1, xyz2, points1, points2, percentage, direction, features):
    del features
    x1_2d = xyz1[0]
    x2_2d = xyz2[0]

    dist = -2.0 * jnp.matmul(xyz1, jnp.swapaxes(xyz2, 1, 2))
    dist = dist + jnp.sum(xyz1 ** 2, -1)[:, :, None]
    dist = dist + jnp.sum(xyz2 ** 2, -1)[:, None, :]
    dists = dist[0]

    dir_cm = jnp.transpose(direction[0], (0, 2, 1)).reshape(S, 3 * K)
    table = jnp.concatenate(
        [x2_2d, percentage[0], dir_cm, points2[0],
         jnp.zeros((S, ROW - 147), jnp.float32)], axis=1)

    idx = _topk_idx(dists)
    idx_flat = jnp.transpose(idx).reshape(B_IDX)
    g = _sc_gather(table, idx_flat)
    g3 = g.reshape(NB, N, ROW)

    interp, wsum = _interpolate(g3, x1_2d)
    scale = jnp.sum(wsum.reshape(1, N)) / N
    coef = 0.3 if N == 4 * S else 0.01
    cs = (1e-08 + coef * scale).reshape(1, 1)
    val = _combine(interp, wsum, points1[0], cs).reshape(1, N, D)
    return val / (jnp.sum(val, axis=-1, keepdims=True) + 1e-09)


# speedup vs baseline: 14.3641x; 1.0410x over previous
import functools

import jax
import jax.numpy as jnp
from jax import lax
from jax.experimental import pallas as pl
from jax.experimental.pallas import tpu as pltpu
from jax.experimental.pallas import tpu_sc as plsc

N = 16384
S = 4096
D = 64
K = 20
NB = 8
GAMMA = 0.5
ROW = 256

T1 = 256
T3 = 256
T4 = 2048

NUM_SC = 2
NUM_SUBCORES = 16
NW = NUM_SC * NUM_SUBCORES
B_IDX = NB * N
B_PER_W = B_IDX // NW
CHUNK = 256


def _topk_body(d_ref, idx_ref, s_ref):
    s_ref[...] = d_ref[...]
    col = lax.broadcasted_iota(jnp.int32, (T1, S), 1)
    for j in range(NB):
        s = s_ref[...]
        am = jnp.argmin(s, axis=1).astype(jnp.int32)[:, None]
        idx_ref[:, j:j + 1] = am
        s_ref[...] = jnp.where(col == am, jnp.inf, s)


def _topk_idx(dists):
    return pl.pallas_call(
        _topk_body,
        grid=(N // T1,),
        in_specs=[pl.BlockSpec((T1, S), lambda i: (i, 0))],
        out_specs=pl.BlockSpec((T1, NB), lambda i: (i, 0)),
        out_shape=jax.ShapeDtypeStruct((N, NB), jnp.int32),
        scratch_shapes=[pltpu.VMEM((T1, S), jnp.float32)],
    )(dists)


def _sc_gather(table, idx_flat):
    mesh = plsc.VectorSubcoreMesh(core_axis_name="c", subcore_axis_name="s")

    @functools.partial(
        pl.kernel,
        mesh=mesh,
        out_type=jax.ShapeDtypeStruct((B_IDX, ROW), jnp.float32),
        scratch_types=[
            pltpu.VMEM((CHUNK,), jnp.int32),
            pltpu.VMEM((CHUNK, ROW), jnp.float32),
            pltpu.SemaphoreType.DMA,
        ],
    )
    def gather_kernel(table_hbm, idx_hbm, out_hbm, idx_v, rows_v, sem):
        wid = lax.axis_index("s") * NUM_SC + lax.axis_index("c")
        base = wid * B_PER_W

        @pl.loop(0, B_PER_W, step=CHUNK)
        def _(c):
            off = base + c
            pltpu.sync_copy(idx_hbm.at[pl.ds(off, CHUNK)], idx_v)
            pltpu.async_copy(table_hbm.at[idx_v], rows_v, sem).wait()
            pltpu.sync_copy(rows_v, out_hbm.at[pl.ds(off, CHUNK)])

    return gather_kernel(table, idx_flat)


def _halve_sum(parts):
    parts = list(parts)
    while len(parts) > 1:
        h = len(parts) // 2
        parts = [parts[i] + parts[h + i] for i in range(h)]
    return parts[0]


def _interp_body(g_ref, x1_ref, interp_ref, wsum_ref):
    x1 = x1_ref[...]
    wk_cols = []
    for n in range(NB):
        g = g_ref[n]
        ux = g[:, 0:1] - x1[:, 0:1]
        uy = g[:, 1:2] - x1[:, 1:2]
        uz = g[:, 2:3] - x1[:, 2:3]
        dist = jnp.sqrt(ux * ux + uy * uy + uz * uz)
        den = dist + 1e-08
        ux, uy, uz = ux / den, uy / den, uz / den
        d0 = g[:, 23:43]
        d1 = g[:, 43:63]
        d2 = g[:, 63:83]
        dn = jnp.sqrt(d0 * d0 + d1 * d1 + d2 * d2) + 1e-08
        simm = (d0 / dn) * ux + (d1 / dn) * uy + (d2 / dn) * uz
        mask = jnp.abs(simm) > GAMMA
        t = jnp.where(mask, g[:, 3:23], 0.0)
        wk_n = t[:, 0:1]
        for k in range(1, K):
            wk_n = wk_n + t[:, k:k + 1]
        wk_cols.append(wk_n)
    wsum = _halve_sum(wk_cols) + 1e-08
    dr_cols = [wk_n / wsum + 1e-06 + 1e-10 for wk_n in wk_cols]
    norm = _halve_sum(dr_cols) + 1e-08
    acc = jnp.zeros((T3, D), jnp.float32)
    for n in range(NB):
        acc = acc + g_ref[n][:, 83:83 + D] * (dr_cols[n] / norm)
    interp_ref[...] = acc
    wsum_ref[...] = wsum


def _interpolate(g3, x1_2d):
    return pl.pallas_call(
        _interp_body,
        grid=(N // T3,),
        in_specs=[
            pl.BlockSpec((NB, T3, ROW), lambda i: (0, i, 0)),
            pl.BlockSpec((T3, 3), lambda i: (i, 0)),
        ],
        out_specs=[
            pl.BlockSpec((T3, D), lambda i: (i, 0)),
            pl.BlockSpec((T3, 1), lambda i: (i, 0)),
        ],
        out_shape=[
            jax.ShapeDtypeStruct((N, D), jnp.float32),
            jax.ShapeDtypeStruct((N, 1), jnp.float32),
        ],
    )(g3, x1_2d)


def _combine_body(interp_ref, wsum_ref, p1_ref, cs_ref, out_ref):
    out_ref[...] = (interp_ref[...] * wsum_ref[...]
                    + cs_ref[0, 0] * p1_ref[...])


def _combine(interp, wsum, p1_2d, cs):
    return pl.pallas_call(
        _combine_body,
        grid=(N // T4,),
        in_specs=[
            pl.BlockSpec((T4, D), lambda i: (i, 0)),
            pl.BlockSpec((T4, 1), lambda i: (i, 0)),
            pl.BlockSpec((T4, D), lambda i: (i, 0)),
            pl.BlockSpec(memory_space=pltpu.SMEM),
        ],
        out_specs=pl.BlockSpec((T4, D), lambda i: (i, 0)),
        out_shape=jax.ShapeDtypeStruct((N, D), jnp.float32),
    )(interp, wsum, p1_2d, cs)


def kernel(xyz1, xyz2, points1, points2, percentage, direction, features):
    del features
    x1_2d = xyz1[0]
    x2_2d = xyz2[0]

    dist = -2.0 * jnp.matmul(xyz1, jnp.swapaxes(xyz2, 1, 2))
    dist = dist + jnp.sum(xyz1 ** 2, -1)[:, :, None]
    dist = dist + jnp.sum(xyz2 ** 2, -1)[:, None, :]
    dists = dist[0]

    dir_cm = jnp.transpose(direction[0], (0, 2, 1)).reshape(S, 3 * K)
    table = jnp.concatenate(
        [x2_2d, percentage[0], dir_cm, points2[0],
         jnp.zeros((S, ROW - 147), jnp.float32)], axis=1)

    idx = _topk_idx(dists)
    idx_flat = jnp.transpose(idx).reshape(B_IDX)
    g = _sc_gather(table, idx_flat)
    g3 = g.reshape(NB, N, ROW)

    interp, wsum = _interpolate(g3, x1_2d)
    scale = jnp.sum(wsum.reshape(1, N)) / N
    coef = 0.3 if N == 4 * S else 0.01
    cs = (1e-08 + coef * scale).reshape(1, 1)
    val = _combine(interp, wsum, points1[0], cs).reshape(1, N, D)
    return val / (jnp.sum(val, axis=-1, keepdims=True) + 1e-09)


# speedup vs baseline: 14.5123x; 1.0103x over previous
import functools

import jax
import jax.numpy as jnp
from jax import lax
from jax.experimental import pallas as pl
from jax.experimental.pallas import tpu as pltpu
from jax.experimental.pallas import tpu_sc as plsc

N = 16384
S = 4096
D = 64
K = 20
NB = 8
GAMMA = 0.5
ROW = 256

T1 = 256
T3 = 256
T4 = 2048

NH = 2
NQ = N // NH
NUM_SC = 2
NUM_SUBCORES = 16
NW = NUM_SC * NUM_SUBCORES
B_IDX = NB * NQ
B_PER_W = B_IDX // NW
CHUNK = 256


def _topk_body(d_ref, idx_ref, s_ref):
    s_ref[...] = d_ref[...]
    col = lax.broadcasted_iota(jnp.int32, (T1, S), 1)
    for j in range(NB):
        s = s_ref[...]
        mn = jnp.min(s, axis=1, keepdims=True)
        am = jnp.min(jnp.where(s <= mn, col, S), axis=1, keepdims=True)
        idx_ref[:, j:j + 1] = am
        s_ref[...] = jnp.where(col == am, jnp.inf, s)


def _topk_idx(dists, half):
    off = half * (NQ // T1)
    return pl.pallas_call(
        _topk_body,
        grid=(NQ // T1,),
        in_specs=[pl.BlockSpec((T1, S), lambda i, o=off: (i + o, 0))],
        out_specs=pl.BlockSpec((T1, NB), lambda i: (i, 0)),
        out_shape=jax.ShapeDtypeStruct((NQ, NB), jnp.int32),
        scratch_shapes=[pltpu.VMEM((T1, S), jnp.float32)],
    )(dists)


def _sc_gather(table, idx_flat):
    mesh = plsc.VectorSubcoreMesh(core_axis_name="c", subcore_axis_name="s")

    @functools.partial(
        pl.kernel,
        mesh=mesh,
        out_type=jax.ShapeDtypeStruct((B_IDX, ROW), jnp.float32),
        scratch_types=[
            pltpu.VMEM((CHUNK,), jnp.int32),
            pltpu.VMEM((CHUNK, ROW), jnp.float32),
            pltpu.SemaphoreType.DMA,
        ],
    )
    def gather_kernel(table_hbm, idx_hbm, out_hbm, idx_v, rows_v, sem):
        wid = lax.axis_index("s") * NUM_SC + lax.axis_index("c")
        base = wid * B_PER_W

        @pl.loop(0, B_PER_W, step=CHUNK)
        def _(c):
            off = base + c
            pltpu.sync_copy(idx_hbm.at[pl.ds(off, CHUNK)], idx_v)
            pltpu.async_copy(table_hbm.at[idx_v], rows_v, sem).wait()
            pltpu.sync_copy(rows_v, out_hbm.at[pl.ds(off, CHUNK)])

    return gather_kernel(table, idx_flat)


def _halve_sum(parts):
    parts = list(parts)
    while len(parts) > 1:
        h = len(parts) // 2
        parts = [parts[i] + parts[h + i] for i in range(h)]
    return parts[0]


def _interp_body(g_ref, x1_ref, interp_ref, wsum_ref):
    x1 = x1_ref[...]
    wk_cols = []
    for n in range(NB):
        g = g_ref[n]
        ux = g[:, 0:1] - x1[:, 0:1]
        uy = g[:, 1:2] - x1[:, 1:2]
        uz = g[:, 2:3] - x1[:, 2:3]
        dist = jnp.sqrt(ux * ux + uy * uy + uz * uz)
        den = dist + 1e-08
        ux, uy, uz = ux / den, uy / den, uz / den
        d0 = g[:, 23:43]
        d1 = g[:, 43:63]
        d2 = g[:, 63:83]
        dn = jnp.sqrt(d0 * d0 + d1 * d1 + d2 * d2) + 1e-08
        simm = (d0 / dn) * ux + (d1 / dn) * uy + (d2 / dn) * uz
        mask = jnp.abs(simm) > GAMMA
        t = jnp.where(mask, g[:, 3:23], 0.0)
        wk_n = t[:, 0:1]
        for k in range(1, K):
            wk_n = wk_n + t[:, k:k + 1]
        wk_cols.append(wk_n)
    wsum = _halve_sum(wk_cols) + 1e-08
    dr_cols = [wk_n / wsum + 1e-06 + 1e-10 for wk_n in wk_cols]
    norm = _halve_sum(dr_cols) + 1e-08
    acc = jnp.zeros((T3, D), jnp.float32)
    for n in range(NB):
        acc = acc + g_ref[n][:, 83:83 + D] * (dr_cols[n] / norm)
    interp_ref[...] = acc
    wsum_ref[...] = wsum


def _interpolate(g3, x1_2d, half):
    off = half * (NQ // T3)
    return pl.pallas_call(
        _interp_body,
        grid=(NQ // T3,),
        in_specs=[
            pl.BlockSpec((NB, T3, ROW), lambda i: (0, i, 0)),
            pl.BlockSpec((T3, 3), lambda i, o=off: (i + o, 0)),
        ],
        out_specs=[
            pl.BlockSpec((T3, D), lambda i: (i, 0)),
            pl.BlockSpec((T3, 1), lambda i: (i, 0)),
        ],
        out_shape=[
            jax.ShapeDtypeStruct((NQ, D), jnp.float32),
            jax.ShapeDtypeStruct((NQ, 1), jnp.float32),
        ],
    )(g3, x1_2d)


def _combine_body(interp_ref, wsum_ref, p1_ref, cs_ref, out_ref):
    out_ref[...] = (interp_ref[...] * wsum_ref[...]
                    + cs_ref[0, 0] * p1_ref[...])


def _combine(interp, wsum, p1_2d, cs):
    return pl.pallas_call(
        _combine_body,
        grid=(N // T4,),
        in_specs=[
            pl.BlockSpec((T4, D), lambda i: (i, 0)),
            pl.BlockSpec((T4, 1), lambda i: (i, 0)),
            pl.BlockSpec((T4, D), lambda i: (i, 0)),
            pl.BlockSpec(memory_space=pltpu.SMEM),
        ],
        out_specs=pl.BlockSpec((T4, D), lambda i: (i, 0)),
        out_shape=jax.ShapeDtypeStruct((N, D), jnp.float32),
    )(interp, wsum, p1_2d, cs)


def kernel(xyz1, xyz2, points1, points2, percentage, direction, features):
    del features
    x1_2d = xyz1[0]
    x2_2d = xyz2[0]

    dist = -2.0 * jnp.matmul(xyz1, jnp.swapaxes(xyz2, 1, 2))
    dist = dist + jnp.sum(xyz1 ** 2, -1)[:, :, None]
    dist = dist + jnp.sum(xyz2 ** 2, -1)[:, None, :]
    dists = dist[0]

    dir_cm = jnp.transpose(direction[0], (0, 2, 1)).reshape(S, 3 * K)
    table = jnp.concatenate(
        [x2_2d, percentage[0], dir_cm, points2[0],
         jnp.zeros((S, ROW - 147), jnp.float32)], axis=1)

    interp_h, wsum_h = [], []
    for h in range(NH):
        idx = _topk_idx(dists, h)
        idx_flat = jnp.transpose(idx).reshape(B_IDX)
        g = _sc_gather(table, idx_flat)
        g3 = g.reshape(NB, NQ, ROW)
        ih, wh = _interpolate(g3, x1_2d, h)
        interp_h.append(ih)
        wsum_h.append(wh)
    interp = jnp.concatenate(interp_h, axis=0)
    wsum = jnp.concatenate(wsum_h, axis=0)
    scale = jnp.sum(wsum.reshape(1, N)) / N
    coef = 0.3 if N == 4 * S else 0.01
    cs = (1e-08 + coef * scale).reshape(1, 1)
    val = _combine(interp, wsum, points1[0], cs).reshape(1, N, D)
    return val / (jnp.sum(val, axis=-1, keepdims=True) + 1e-09)


# speedup vs baseline: 15.1501x; 1.0439x over previous
import functools

import jax
import jax.numpy as jnp
from jax import lax
from jax.experimental import pallas as pl
from jax.experimental.pallas import tpu as pltpu
from jax.experimental.pallas import tpu_sc as plsc

N = 16384
S = 4096
D = 64
K = 20
NB = 8
GAMMA = 0.5
ROW = 256

T1 = 256
T3 = 256
T4 = 2048

NH = 2
NQ = N // NH
NUM_SC = 2
NUM_SUBCORES = 16
NW = NUM_SC * NUM_SUBCORES
B_IDX = NB * NQ
B_PER_W = B_IDX // NW
CHUNK = 256


def _topk_body(d_ref, idx_ref, s_ref):
    s_ref[...] = d_ref[...]
    col = lax.broadcasted_iota(jnp.int32, (T1, S), 1)
    for j in range(NB):
        s = s_ref[...]
        am = jnp.argmin(s, axis=1).astype(jnp.int32)[:, None]
        idx_ref[:, j:j + 1] = am
        s_ref[...] = jnp.where(col == am, jnp.inf, s)


def _topk_idx(dists, half):
    off = half * (NQ // T1)
    return pl.pallas_call(
        _topk_body,
        grid=(NQ // T1,),
        in_specs=[pl.BlockSpec((T1, S), lambda i, o=off: (i + o, 0))],
        out_specs=pl.BlockSpec((T1, NB), lambda i: (i, 0)),
        out_shape=jax.ShapeDtypeStruct((NQ, NB), jnp.int32),
        scratch_shapes=[pltpu.VMEM((T1, S), jnp.float32)],
    )(dists)


def _sc_gather(table, idx_flat):
    mesh = plsc.VectorSubcoreMesh(core_axis_name="c", subcore_axis_name="s")

    @functools.partial(
        pl.kernel,
        mesh=mesh,
        out_type=jax.ShapeDtypeStruct((B_IDX, ROW), jnp.float32),
        scratch_types=[
            pltpu.VMEM((CHUNK,), jnp.int32),
            pltpu.VMEM((CHUNK, ROW), jnp.float32),
            pltpu.SemaphoreType.DMA,
        ],
    )
    def gather_kernel(table_hbm, idx_hbm, out_hbm, idx_v, rows_v, sem):
        wid = lax.axis_index("s") * NUM_SC + lax.axis_index("c")
        base = wid * B_PER_W

        @pl.loop(0, B_PER_W, step=CHUNK)
        def _(c):
            off = base + c
            pltpu.sync_copy(idx_hbm.at[pl.ds(off, CHUNK)], idx_v)
            pltpu.async_copy(table_hbm.at[idx_v], rows_v, sem).wait()
            pltpu.sync_copy(rows_v, out_hbm.at[pl.ds(off, CHUNK)])

    return gather_kernel(table, idx_flat)


def _halve_sum(parts):
    parts = list(parts)
    while len(parts) > 1:
        h = len(parts) // 2
        parts = [parts[i] + parts[h + i] for i in range(h)]
    return parts[0]


def _interp_body(g_ref, x1_ref, interp_ref, wsum_ref):
    x1 = x1_ref[...]
    wk_cols = []
    for n in range(NB):
        g = g_ref[n]
        ux = g[:, 0:1] - x1[:, 0:1]
        uy = g[:, 1:2] - x1[:, 1:2]
        uz = g[:, 2:3] - x1[:, 2:3]
        dist = jnp.sqrt(ux * ux + uy * uy + uz * uz)
        den = dist + 1e-08
        ux, uy, uz = ux / den, uy / den, uz / den
        d0 = g[:, 23:43]
        d1 = g[:, 43:63]
        d2 = g[:, 63:83]
        dn = jnp.sqrt(d0 * d0 + d1 * d1 + d2 * d2) + 1e-08
        simm = (d0 / dn) * ux + (d1 / dn) * uy + (d2 / dn) * uz
        mask = jnp.abs(simm) > GAMMA
        t = jnp.where(mask, g[:, 3:23], 0.0)
        wk_n = t[:, 0:1]
        for k in range(1, K):
            wk_n = wk_n + t[:, k:k + 1]
        wk_cols.append(wk_n)
    wsum = _halve_sum(wk_cols) + 1e-08
    dr_cols = [wk_n / wsum + 1e-06 + 1e-10 for wk_n in wk_cols]
    norm = _halve_sum(dr_cols) + 1e-08
    acc = jnp.zeros((T3, D), jnp.float32)
    for n in range(NB):
        acc = acc + g_ref[n][:, 83:83 + D] * (dr_cols[n] / norm)
    interp_ref[...] = acc
    wsum_ref[...] = wsum


def _interpolate(g3, x1_2d, half):
    off = half * (NQ // T3)
    return pl.pallas_call(
        _interp_body,
        grid=(NQ // T3,),
        in_specs=[
            pl.BlockSpec((NB, T3, ROW), lambda i: (0, i, 0)),
            pl.BlockSpec((T3, 3), lambda i, o=off: (i + o, 0)),
        ],
        out_specs=[
            pl.BlockSpec((T3, D), lambda i: (i, 0)),
            pl.BlockSpec((T3, 1), lambda i: (i, 0)),
        ],
        out_shape=[
            jax.ShapeDtypeStruct((NQ, D), jnp.float32),
            jax.ShapeDtypeStruct((NQ, 1), jnp.float32),
        ],
    )(g3, x1_2d)


def _combine_body(interp_ref, wsum_ref, p1_ref, cs_ref, out_ref):
    out_ref[...] = (interp_ref[...] * wsum_ref[...]
                    + cs_ref[0, 0] * p1_ref[...])


def _combine(interp, wsum, p1_2d, cs):
    return pl.pallas_call(
        _combine_body,
        grid=(N // T4,),
        in_specs=[
            pl.BlockSpec((T4, D), lambda i: (i, 0)),
            pl.BlockSpec((T4, 1), lambda i: (i, 0)),
            pl.BlockSpec((T4, D), lambda i: (i, 0)),
            pl.BlockSpec(memory_space=pltpu.SMEM),
        ],
        out_specs=pl.BlockSpec((T4, D), lambda i: (i, 0)),
        out_shape=jax.ShapeDtypeStruct((N, D), jnp.float32),
    )(interp, wsum, p1_2d, cs)


def kernel(xyz1, xyz2, points1, points2, percentage, direction, features):
    del features
    x1_2d = xyz1[0]
    x2_2d = xyz2[0]

    dist = -2.0 * jnp.matmul(xyz1, jnp.swapaxes(xyz2, 1, 2))
    dist = dist + jnp.sum(xyz1 ** 2, -1)[:, :, None]
    dist = dist + jnp.sum(xyz2 ** 2, -1)[:, None, :]
    dists = dist[0]

    dir_cm = jnp.transpose(direction[0], (0, 2, 1)).reshape(S, 3 * K)
    table = jnp.concatenate(
        [x2_2d, percentage[0], dir_cm, points2[0],
         jnp.zeros((S, ROW - 147), jnp.float32)], axis=1)

    interp_h, wsum_h = [], []
    for h in range(NH):
        idx = _topk_idx(dists, h)
        idx_flat = jnp.transpose(idx).reshape(B_IDX)
        g = _sc_gather(table, idx_flat)
        g3 = g.reshape(NB, NQ, ROW)
        ih, wh = _interpolate(g3, x1_2d, h)
        interp_h.append(ih)
        wsum_h.append(wh)
    interp = jnp.concatenate(interp_h, axis=0)
    wsum = jnp.concatenate(wsum_h, axis=0)
    scale = jnp.sum(wsum.reshape(1, N)) / N
    coef = 0.3 if N == 4 * S else 0.01
    cs = (1e-08 + coef * scale).reshape(1, 1)
    val = _combine(interp, wsum, points1[0], cs).reshape(1, N, D)
    return val / (jnp.sum(val, axis=-1, keepdims=True) + 1e-09)
